# baseline (speedup 1.0000x reference)
import jax
import jax.numpy as jnp
from jax import lax
from jax.experimental import pallas as pl

N = 512
D = 128
H1 = 4
C1 = 64
HID = H1 * C1
C2 = 128


def _softmax_factors(a_srcT, a_dst):
    s_max = jnp.max(a_srcT, axis=1, keepdims=True)
    dps = a_dst + jnp.transpose(s_max)
    m = jnp.maximum(dps, 0.2 * dps)
    cu = jnp.exp(a_dst - m)
    cv = jnp.exp(0.2 * a_dst - m)
    ru = jnp.exp(a_srcT)
    rv = jnp.exp(0.2 * a_srcT)
    return cu, cv, ru, rv


def _gat_dense(h_ext, cu, cv, ru, rv):
    p = jnp.maximum(cu * ru, cv * rv)
    acc = jnp.dot(p, h_ext, preferred_element_type=jnp.float32)
    c = h_ext.shape[1] - 1
    return acc[:, :c] / (acc[:, c:] + 1e-16)


def _blockdiag(att_row, heads, ch):
    att_col = jnp.transpose(att_row)
    if heads == 1:
        return att_col
    r = lax.broadcasted_iota(jnp.int32, (heads * ch, heads), 0) // ch
    c = lax.broadcasted_iota(jnp.int32, (heads * ch, heads), 1)
    return jnp.where(r == c, att_col, 0.0)


def _encoder_kernel(x_ref, W1_ref, as1_ref, ad1_ref, b1_ref,
                    W2_ref, as2_ref, ad2_ref, b2_ref, out_ref):
    x = x_ref[...]
    h1 = jnp.dot(x, W1_ref[...], preferred_element_type=jnp.float32)

    ones = jnp.ones((N, 1), dtype=jnp.float32)
    h_exts = [jnp.concatenate([h1[:, hd * C1:(hd + 1) * C1], ones], axis=1)
              for hd in range(H1)]

    As1 = _blockdiag(as1_ref[...], H1, C1)
    Ad1 = _blockdiag(ad1_ref[...], H1, C1)
    a_dst = jnp.dot(h1, Ad1, preferred_element_type=jnp.float32)
    a_srcT = lax.dot_general(As1, h1, (((0,), (1,)), ((), ())),
                             preferred_element_type=jnp.float32)

    cu, cv, ru, rv = _softmax_factors(a_srcT, a_dst)

    outs = []
    for hd in range(H1):
        outs.append(_gat_dense(h_exts[hd],
                               cu[:, hd:hd + 1], cv[:, hd:hd + 1],
                               ru[hd:hd + 1, :], rv[hd:hd + 1, :]))
    o1 = jnp.concatenate(outs, axis=1) + b1_ref[...]
    o1 = jnp.maximum(o1, 0.0)

    h2 = jnp.dot(o1, W2_ref[...], preferred_element_type=jnp.float32)
    h2_ext = jnp.concatenate([h2, ones], axis=1)
    ad2_col = _blockdiag(ad2_ref[...], 1, C2)
    as2_col = _blockdiag(as2_ref[...], 1, C2)
    a_dst2 = jnp.dot(h2, ad2_col, preferred_element_type=jnp.float32)
    a_src2T = lax.dot_general(as2_col, h2, (((0,), (1,)), ((), ())),
                              preferred_element_type=jnp.float32)
    cu2, cv2, ru2, rv2 = _softmax_factors(a_src2T, a_dst2)
    o2 = _gat_dense(h2_ext, cu2, cv2, ru2, rv2) + b2_ref[...]
    out_ref[...] = o2


def kernel(x, W1, att_src1, att_dst1, b1, W2, att_src2, att_dst2, b2):
    return pl.pallas_call(
        _encoder_kernel,
        out_shape=jax.ShapeDtypeStruct((N, C2), jnp.float32),
    )(x, W1,
      att_src1.reshape(1, HID), att_dst1.reshape(1, HID), b1.reshape(1, HID),
      W2,
      att_src2.reshape(1, C2), att_dst2.reshape(1, C2), b2.reshape(1, C2))
